# baseline (speedup 1.0000x reference)
import functools

import jax
import jax.numpy as jnp
from jax import lax
from jax.experimental import pallas as pl
from jax.experimental.pallas import tpu as pltpu
from jax.experimental.pallas import tpu_sc as plsc

B = 16384
N_USER = 1000000
D_U = 32
N_CAT = 1000
D_C = 16
D_OUT = D_U + D_C + 2

NC = 2
NS = 16
NW = NC * NS
BPW = B // NW
CH = 128
NCH = BPW // CH
SLABW = 128
UPS = SLABW // D_U
L = 16


def _sc_body(uid_hbm, cid_hbm, wu_hbm, wc_hbm,
             outu_hbm, outc_hbm,
             uid_v, cid_v, qid_v, rows4_v, wc_v, outu_v, outc_v, sem_g):
    wid = lax.axis_index("s") * NC + lax.axis_index("c")
    base = wid * BPW

    pltpu.sync_copy(uid_hbm.at[pl.ds(base, BPW)], uid_v)
    pltpu.sync_copy(cid_hbm.at[pl.ds(base, BPW)], cid_v)
    cat_stage = pltpu.async_copy(wc_hbm, wc_v, sem_g)

    def qid_body(g, carry):
        u16 = uid_v[pl.ds(g * L, L)]
        qid_v[pl.ds(g * L, L)] = u16 >> UPS.bit_length() - 1
        return carry

    lax.fori_loop(0, BPW // L, qid_body, 0, unroll=8)

    gathers = []
    for j in range(NCH):
        gathers.append(pltpu.async_copy(
            wu_hbm.at[qid_v.at[pl.ds(j * CH, CH)]],
            rows4_v.at[pl.ds(j * CH, CH)], sem_g))

    cat_stage.wait()

    def cat_body(g, carry):
        c16 = cid_v[pl.ds(g * L, L)] * D_C
        for r in range(L):
            b = g * L + r
            outc_v[pl.ds(b * D_C, D_C)] = wc_v[pl.ds(c16[r], D_C)]
        return carry

    lax.fori_loop(0, BPW // L, cat_body, 0, unroll=2)

    for g in gathers:
        g.wait()

    def user_body(g, carry):
        o16 = (uid_v[pl.ds(g * L, L)] & (UPS - 1)) * D_U
        for r in range(L):
            b = g * L + r
            off = o16[r]
            outu_v[pl.ds(b * D_U, L)] = rows4_v[b, pl.ds(off, L)]
            outu_v[pl.ds(b * D_U + L, L)] = rows4_v[b, pl.ds(off + L, L)]
        return carry

    lax.fori_loop(0, BPW // L, user_body, 0, unroll=2)

    pltpu.sync_copy(outu_v, outu_hbm.at[pl.ds(base * D_U, BPW * D_U)])
    pltpu.sync_copy(outc_v, outc_hbm.at[pl.ds(base * D_C, BPW * D_C)])


def kernel(user_id, category, day_sin, day_cos, W_user, W_category):
    mesh = plsc.VectorSubcoreMesh(core_axis_name="c", subcore_axis_name="s")
    run = pl.kernel(
        _sc_body, mesh=mesh,
        compiler_params=pltpu.CompilerParams(needs_layout_passes=False),
        out_type=(jax.ShapeDtypeStruct((B * D_U,), jnp.float32),
                  jax.ShapeDtypeStruct((B * D_C,), jnp.float32)),
        scratch_types=[
            pltpu.VMEM((BPW,), jnp.int32),
            pltpu.VMEM((BPW,), jnp.int32),
            pltpu.VMEM((BPW,), jnp.int32),
            pltpu.VMEM((BPW, SLABW), jnp.float32),
            pltpu.VMEM((N_CAT * D_C,), jnp.float32),
            pltpu.VMEM((BPW * D_U,), jnp.float32),
            pltpu.VMEM((BPW * D_C,), jnp.float32),
            pltpu.SemaphoreType.DMA,
        ],
    )
    eu, ec = run(user_id, category,
                 W_user.reshape(N_USER // UPS, SLABW),
                 W_category.reshape(N_CAT * D_C))
    return jnp.concatenate([eu.reshape(B, D_U), ec.reshape(B, D_C),
                            day_sin, day_cos], axis=1)
